# baseline (speedup 1.0000x reference)
import functools

import jax
import jax.numpy as jnp
from jax import lax
from jax.experimental import pallas as pl
from jax.experimental.pallas import tpu as pltpu
from jax.experimental.pallas import tpu_sc as plsc

_NC = 2
_NS = 16


def _sc_gather_t(tabT3, xTflat, B):
    F, D, V = tabT3.shape
    L = 128
    half = 64
    hb = half * L
    nb2 = B // hb
    R = F * D
    nw = _NC * _NS
    per_w = R // nw
    assert per_w * nw == R and nb2 * hb == B

    mesh = plsc.VectorSubcoreMesh(core_axis_name="c", subcore_axis_name="s")

    @functools.partial(
        pl.kernel,
        out_type=jax.ShapeDtypeStruct((R // 8, B // L, 8, L), jnp.float32),
        mesh=mesh,
        scratch_types=[
            pltpu.VMEM((V,), jnp.float32),
            pltpu.VMEM((hb,), jnp.int32),
            pltpu.VMEM((half, L), jnp.float32),
            pltpu.SemaphoreType.DMA,
        ],
        compiler_params=pltpu.CompilerParams(
            needs_layout_passes=False),
    )
    def gk(tab_hbm, x_hbm, out_hbm, row_v, idx_v, out_v, sem):
        wid = lax.axis_index("s") * _NC + lax.axis_index("c")

        def row_body(k, carry):
            r = wid * per_w + k
            f = r // D
            d = lax.rem(r, D)
            rt = r // 8
            rs = lax.rem(r, 8)
            pltpu.sync_copy(tab_hbm.at[f, d, :], row_v)

            def half_body(h, carry2):
                pltpu.sync_copy(x_hbm.at[pl.ds(f * B + h * hb, hb)], idx_v)

                def gather16(m, carry3):
                    for l in range(8):
                        vv = idx_v[pl.ds(m * L + l * 16, 16)]
                        vals = plsc.load_gather(row_v, [vv])
                        out_v[m, pl.ds(l * 16, 16)] = vals
                    return carry3

                lax.fori_loop(0, half, gather16, 0, unroll=4)
                pltpu.sync_copy(
                    out_v, out_hbm.at[rt, pl.ds(h * half, half), rs, :])
                return carry2

            lax.fori_loop(0, nb2, half_body, 0)
            return carry

        lax.fori_loop(0, per_w, row_body, 0)

    return gk(tabT3, xTflat)


def _tc_towers_t(emb4, smat, params, cbb):
    nrt, nct, _, L = emb4.shape
    Bn = nct * L
    grid = nct // cbb

    def body(emb_ref, smat_ref,
             cw1, cb1, cw2, cb2, cw3, cb3,
             tw1, tb1, tw2, tb2, tw3, tb3, out_ref):
        parts = []
        for j in range(cbb):
            parts.append(jnp.reshape(emb_ref[:, j, :, :], (nrt * 8, L)))
        e = jnp.concatenate(parts, axis=1) if cbb > 1 else parts[0]
        cn = (((0,), (0,)), ((), ()))
        s = lax.dot_general(smat_ref[...], e, cn,
                            preferred_element_type=jnp.float32)
        ss = jnp.sum(s * s, axis=0, keepdims=True)
        sq = jnp.sum(e * e, axis=0, keepdims=True)
        fm = 0.5 * (ss - sq)
        outs = []
        for (w1, b1, w2, b2, w3, b3) in (
                (cw1, cb1, cw2, cb2, cw3, cb3),
                (tw1, tb1, tw2, tb2, tw3, tb3)):
            h = jnp.maximum(lax.dot_general(
                w1[...], e, cn, preferred_element_type=jnp.float32)
                + b1[...], 0.0)
            h = jnp.maximum(lax.dot_general(
                w2[...], h, cn, preferred_element_type=jnp.float32)
                + b2[...], 0.0)
            deep = lax.dot_general(
                w3[...], h, cn, preferred_element_type=jnp.float32) + b3[...]
            z = deep + fm
            outs.append(1.0 / (1.0 + jnp.exp(-z)))
        cvr, ctr = outs
        res = jnp.concatenate([cvr, ctr, cvr * ctr], axis=0)
        res = jnp.clip(res, 1e-15, 1.0 - 1e-15)
        out_ref[...] = jnp.transpose(res, (1, 0))

    full = lambda shape: pl.BlockSpec(shape, lambda i: (0,) * len(shape))
    in_specs = [pl.BlockSpec((nrt, cbb, 8, L), lambda i: (0, i, 0, 0)),
                full(smat.shape)]
    in_specs += [full(p.shape) for p in params]

    return pl.pallas_call(
        body,
        grid=(grid,),
        in_specs=in_specs,
        out_specs=pl.BlockSpec((cbb * L, 3), lambda i: (i, 0)),
        out_shape=jax.ShapeDtypeStruct((Bn, 3), jnp.float32),
    )(emb4, smat, *params)


def kernel(x, tables, cvr_w1, cvr_b1, cvr_w2, cvr_b2, cvr_w3, cvr_b3,
           ctr_w1, ctr_b1, ctr_w2, ctr_b2, ctr_w3, ctr_b3):
    F, V, D = tables.shape
    B = x.shape[0]
    tabT3 = jnp.transpose(tables, (0, 2, 1))
    xTflat = jnp.transpose(x, (1, 0)).reshape(-1)
    emb4 = _sc_gather_t(tabT3, xTflat, B)

    din = F * D
    smat = (jnp.arange(din, dtype=jnp.int32)[:, None] % D
            == jnp.arange(D, dtype=jnp.int32)[None, :]).astype(jnp.float32)
    col = lambda b: b[:, None]
    params = (cvr_w1, col(cvr_b1), cvr_w2, col(cvr_b2), cvr_w3, col(cvr_b3),
              ctr_w1, col(ctr_b1), ctr_w2, col(ctr_b2), ctr_w3, col(ctr_b3))
    return _tc_towers_t(emb4, smat, params, cbb=4)


# speedup vs baseline: 1.2910x; 1.2910x over previous
import functools

import jax
import jax.numpy as jnp
from jax import lax
from jax.experimental import pallas as pl
from jax.experimental.pallas import tpu as pltpu
from jax.experimental.pallas import tpu_sc as plsc

_NC = 2
_NS = 16


def _sc_gather_t(tabT3, xTflat, B):
    F, D, V = tabT3.shape
    L = 128
    half = 64
    hb = half * L
    nb2 = B // hb
    R = F * D
    nw = _NC * _NS
    per_w = R // nw
    assert per_w * nw == R and nb2 * hb == B

    mesh = plsc.VectorSubcoreMesh(core_axis_name="c", subcore_axis_name="s")

    @functools.partial(
        pl.kernel,
        out_type=jax.ShapeDtypeStruct((R // 8, B // L, 8, L), jnp.float32),
        mesh=mesh,
        scratch_types=[
            pltpu.VMEM((V,), jnp.float32),
            pltpu.VMEM((hb,), jnp.int32),
            pltpu.VMEM((half, L), jnp.float32),
            pltpu.VMEM((half, L), jnp.float32),
            pltpu.SemaphoreType.DMA,
            pltpu.SemaphoreType.DMA,
        ],
        compiler_params=pltpu.CompilerParams(
            needs_layout_passes=False),
    )
    def gk(tab_hbm, x_hbm, out_hbm, row_v, idx_v, out_v0, out_v1, w0, w1):
        wid = lax.axis_index("s") * _NC + lax.axis_index("c")

        def row_body(k, carry):
            r = wid * per_w + k
            f = r // D
            d = lax.rem(r, D)
            rt = r // 8
            rs = lax.rem(r, 8)
            pltpu.sync_copy(tab_hbm.at[f, d, :], row_v)

            for h, (out_v, wsem) in enumerate(((out_v0, w0), (out_v1, w1))):
                pltpu.sync_copy(x_hbm.at[pl.ds(f * B + h * hb, hb)], idx_v)

                @pl.when(k > 0)
                def _():
                    pltpu.make_async_copy(
                        out_hbm.at[0, pl.ds(0, half), 0, :], out_v,
                        wsem).wait()

                def gather16(m, carry3):
                    for l in range(8):
                        vv = idx_v[pl.ds(m * L + l * 16, 16)]
                        vals = plsc.load_gather(row_v, [vv])
                        out_v[m, pl.ds(l * 16, 16)] = vals
                    return carry3

                lax.fori_loop(0, half, gather16, 0)
                pltpu.async_copy(
                    out_v, out_hbm.at[rt, pl.ds(h * half, half), rs, :], wsem)
            return carry

        lax.fori_loop(0, per_w, row_body, 0)
        for out_v, wsem in ((out_v0, w0), (out_v1, w1)):
            pltpu.make_async_copy(
                out_hbm.at[0, pl.ds(0, half), 0, :], out_v, wsem).wait()

    return gk(tabT3, xTflat)


def _tc_towers_t(emb4, smat, params, cbb):
    nrt, nct, _, L = emb4.shape
    Bn = nct * L
    grid = nct // cbb

    def body(emb_ref, smat_ref,
             cw1, cb1, cw2, cb2, cw3, cb3,
             tw1, tb1, tw2, tb2, tw3, tb3, out_ref):
        parts = []
        for j in range(cbb):
            parts.append(jnp.reshape(emb_ref[:, j, :, :], (nrt * 8, L)))
        e = jnp.concatenate(parts, axis=1) if cbb > 1 else parts[0]
        cn = (((0,), (0,)), ((), ()))
        s = lax.dot_general(smat_ref[...], e, cn,
                            preferred_element_type=jnp.float32)
        ss = jnp.sum(s * s, axis=0, keepdims=True)
        sq = jnp.sum(e * e, axis=0, keepdims=True)
        fm = 0.5 * (ss - sq)
        outs = []
        for (w1, b1, w2, b2, w3, b3) in (
                (cw1, cb1, cw2, cb2, cw3, cb3),
                (tw1, tb1, tw2, tb2, tw3, tb3)):
            h = jnp.maximum(lax.dot_general(
                w1[...], e, cn, preferred_element_type=jnp.float32)
                + b1[...], 0.0)
            h = jnp.maximum(lax.dot_general(
                w2[...], h, cn, preferred_element_type=jnp.float32)
                + b2[...], 0.0)
            deep = lax.dot_general(
                w3[...], h, cn, preferred_element_type=jnp.float32) + b3[...]
            z = deep + fm
            outs.append(1.0 / (1.0 + jnp.exp(-z)))
        cvr, ctr = outs
        res = jnp.concatenate([cvr, ctr, cvr * ctr], axis=0)
        res = jnp.clip(res, 1e-15, 1.0 - 1e-15)
        out_ref[...] = jnp.transpose(res, (1, 0))

    full = lambda shape: pl.BlockSpec(shape, lambda i: (0,) * len(shape))
    in_specs = [pl.BlockSpec((nrt, cbb, 8, L), lambda i: (0, i, 0, 0)),
                full(smat.shape)]
    in_specs += [full(p.shape) for p in params]

    return pl.pallas_call(
        body,
        grid=(grid,),
        in_specs=in_specs,
        out_specs=pl.BlockSpec((cbb * L, 3), lambda i: (i, 0)),
        out_shape=jax.ShapeDtypeStruct((Bn, 3), jnp.float32),
    )(emb4, smat, *params)


def kernel(x, tables, cvr_w1, cvr_b1, cvr_w2, cvr_b2, cvr_w3, cvr_b3,
           ctr_w1, ctr_b1, ctr_w2, ctr_b2, ctr_w3, ctr_b3):
    F, V, D = tables.shape
    B = x.shape[0]
    tabT3 = jnp.transpose(tables, (0, 2, 1))
    xTflat = jnp.transpose(x, (1, 0)).reshape(-1)
    emb4 = _sc_gather_t(tabT3, xTflat, B)

    din = F * D
    smat = (jnp.arange(din, dtype=jnp.int32)[:, None] % D
            == jnp.arange(D, dtype=jnp.int32)[None, :]).astype(jnp.float32)
    col = lambda b: b[:, None]
    params = (cvr_w1, col(cvr_b1), cvr_w2, col(cvr_b2), cvr_w3, col(cvr_b3),
              ctr_w1, col(ctr_b1), ctr_w2, col(ctr_b2), ctr_w3, col(ctr_b3))
    return _tc_towers_t(emb4, smat, params, cbb=4)


# speedup vs baseline: 1.3952x; 1.0807x over previous
import functools

import jax
import jax.numpy as jnp
from jax import lax
from jax.experimental import pallas as pl
from jax.experimental.pallas import tpu as pltpu
from jax.experimental.pallas import tpu_sc as plsc

_NC = 2
_NS = 16


def _sc_gather_t(tabT3, xTflat, B):
    F, D, V = tabT3.shape
    L = 128
    half = 64
    hb = half * L
    nb2 = B // hb
    R = F * D
    nw = _NC * _NS
    per_w = R // nw
    assert per_w * nw == R and nb2 * hb == B

    mesh = plsc.VectorSubcoreMesh(core_axis_name="c", subcore_axis_name="s")

    @functools.partial(
        pl.kernel,
        out_type=jax.ShapeDtypeStruct((R // 8, B // L, 8, L), jnp.float32),
        mesh=mesh,
        scratch_types=[
            pltpu.VMEM((V,), jnp.float32),
            pltpu.VMEM((hb,), jnp.int32),
            pltpu.VMEM((half, L), jnp.float32),
            pltpu.VMEM((half, L), jnp.float32),
            pltpu.SemaphoreType.DMA,
            pltpu.SemaphoreType.DMA,
        ],
        compiler_params=pltpu.CompilerParams(
            needs_layout_passes=False),
    )
    def gk(tab_hbm, x_hbm, out_hbm, row_v, idx_v, out_v0, out_v1, w0, w1):
        wid = lax.axis_index("s") * _NC + lax.axis_index("c")

        def row_body(k, carry):
            r = wid * per_w + k
            f = r // D
            d = lax.rem(r, D)
            rt = r // 8
            rs = lax.rem(r, 8)
            pltpu.sync_copy(tab_hbm.at[f, d, :], row_v)

            for h, (out_v, wsem) in enumerate(((out_v0, w0), (out_v1, w1))):
                pltpu.sync_copy(x_hbm.at[pl.ds(f * B + h * hb, hb)], idx_v)

                @pl.when(k > 0)
                def _():
                    pltpu.make_async_copy(
                        out_hbm.at[0, pl.ds(0, half), 0, :], out_v,
                        wsem).wait()

                def gather16(m, carry3):
                    for l in range(8):
                        vv = idx_v[pl.ds(m * L + l * 16, 16)]
                        vals = plsc.load_gather(row_v, [vv])
                        out_v[m, pl.ds(l * 16, 16)] = vals
                    return carry3

                lax.fori_loop(0, half, gather16, 0)
                pltpu.async_copy(
                    out_v, out_hbm.at[rt, pl.ds(h * half, half), rs, :], wsem)
            return carry

        lax.fori_loop(0, per_w, row_body, 0)
        for out_v, wsem in ((out_v0, w0), (out_v1, w1)):
            pltpu.make_async_copy(
                out_hbm.at[0, pl.ds(0, half), 0, :], out_v, wsem).wait()

    return gk(tabT3, xTflat)


def _tc_towers_t(emb4, smat, params, cbb):
    nrt, nct, _, L = emb4.shape
    Bn = nct * L
    grid = nct // cbb

    def body(emb_ref, smat_ref,
             cw1, cb1, cw2, cb2, cw3, cb3,
             tw1, tb1, tw2, tb2, tw3, tb3, out_ref):
        parts = []
        for j in range(cbb):
            parts.append(jnp.reshape(emb_ref[:, j, :, :], (nrt * 8, L)))
        e = jnp.concatenate(parts, axis=1) if cbb > 1 else parts[0]
        cn = (((0,), (0,)), ((), ()))
        s = lax.dot_general(smat_ref[...], e, cn,
                            preferred_element_type=jnp.float32)
        ss = jnp.sum(s * s, axis=0, keepdims=True)
        sq = jnp.sum(e * e, axis=0, keepdims=True)
        fm = 0.5 * (ss - sq)
        outs = []
        for (w1, b1, w2, b2, w3, b3) in (
                (cw1, cb1, cw2, cb2, cw3, cb3),
                (tw1, tb1, tw2, tb2, tw3, tb3)):
            h = jnp.maximum(lax.dot_general(
                w1[...], e, cn, preferred_element_type=jnp.float32)
                + b1[...], 0.0)
            h = jnp.maximum(lax.dot_general(
                w2[...], h, cn, preferred_element_type=jnp.float32)
                + b2[...], 0.0)
            deep = lax.dot_general(
                w3[...], h, cn, preferred_element_type=jnp.float32) + b3[...]
            z = deep + fm
            outs.append(1.0 / (1.0 + jnp.exp(-z)))
        cvr, ctr = outs
        res = jnp.concatenate([cvr, ctr, cvr * ctr], axis=0)
        res = jnp.clip(res, 1e-15, 1.0 - 1e-15)
        out_ref[...] = jnp.transpose(res, (1, 0))

    full = lambda shape: pl.BlockSpec(shape, lambda i: (0,) * len(shape))
    in_specs = [pl.BlockSpec((nrt, cbb, 8, L), lambda i: (0, i, 0, 0)),
                full(smat.shape)]
    in_specs += [full(p.shape) for p in params]

    return pl.pallas_call(
        body,
        grid=(grid,),
        in_specs=in_specs,
        out_specs=pl.BlockSpec((cbb * L, 3), lambda i: (i, 0)),
        out_shape=jax.ShapeDtypeStruct((Bn, 3), jnp.float32),
    )(emb4, smat, *params)


def kernel(x, tables, cvr_w1, cvr_b1, cvr_w2, cvr_b2, cvr_w3, cvr_b3,
           ctr_w1, ctr_b1, ctr_w2, ctr_b2, ctr_w3, ctr_b3):
    F, V, D = tables.shape
    B = x.shape[0]
    tabT3 = jnp.transpose(tables, (0, 2, 1))
    xTflat = jnp.transpose(x, (1, 0)).reshape(-1)
    emb4 = _sc_gather_t(tabT3, xTflat, B)

    din = F * D
    smat = (jnp.arange(din, dtype=jnp.int32)[:, None] % D
            == jnp.arange(D, dtype=jnp.int32)[None, :]).astype(jnp.float32)
    col = lambda b: b[:, None]
    params = (cvr_w1, col(cvr_b1), cvr_w2, col(cvr_b2), cvr_w3, col(cvr_b3),
              ctr_w1, col(ctr_b1), ctr_w2, col(ctr_b2), ctr_w3, col(ctr_b3))
    return _tc_towers_t(emb4, smat, params, cbb=8)
